# baseline (speedup 1.0000x reference)
import functools

import jax
import jax.numpy as jnp
from jax import lax
from jax.experimental import pallas as pl
from jax.experimental.pallas import tpu as pltpu
from jax.experimental.pallas import tpu_sc as plsc

_NC = 2
_NS = 16
_NW = _NC * _NS
_L = 16

_D = 4
_NCHUNKS = 16


def _body(x_ref, w_ref, out_ref, wt_v, i0, i1, o0, o1, si0, si1, so0, so1,
          cn):
    wid = lax.axis_index("s") * _NC + lax.axis_index("c")
    base = wid * _NCHUNKS * cn

    pltpu.sync_copy(w_ref, wt_v)
    for k in range(3):
        wt_v[pl.ds(k * _L, _L)] = wt_v[pl.ds(k * _L, _L)] * 0.5 + 1.0

    lane = lax.iota(jnp.int32, _L)

    ibufs = (i0, i1)
    obufs = (o0, o1)
    isems = (si0, si1)
    osems = (so0, so1)

    def in_copy(c, b):
        return pltpu.make_async_copy(
            x_ref.at[pl.ds(base + c * cn, cn)], ibufs[b], isems[b])

    def out_copy(c, b):
        return pltpu.make_async_copy(
            obufs[b], out_ref.at[pl.ds((base + c * cn) * _D, cn * _D)],
            osems[b])

    def compute(b):
        iv = ibufs[b]
        ov = obufs[b]

        @plsc.parallel_loop(0, cn // _L, unroll=8)
        def _(i):
            p = i * _L + lane
            idx16 = plsc.load_gather(iv, [p])
            off = idx16 * _D
            q = p * _D
            for c in range(_D):
                g = plsc.load_gather(wt_v, [off + c])
                plsc.store_scatter(ov, [q + c], g)

    in_copy(0, 0).start()
    in_copy(1, 1).start()

    def superstep(st, _):
        for b in range(2):
            c = 2 * st + b
            in_copy(c, b).wait()

            @pl.when(st > 0)
            def _():
                out_copy(c - 2, b).wait()

            compute(b)
            out_copy(c, b).start()

            @pl.when(c + 2 < _NCHUNKS)
            def _():
                in_copy(c + 2, b).start()

        return 0

    lax.fori_loop(0, _NCHUNKS // 2, superstep, 0)
    out_copy(_NCHUNKS - 2, 0).wait()
    out_copy(_NCHUNKS - 1, 1).wait()


@functools.partial(jax.jit, static_argnames=("n",))
def _lookup(x, wf, n):
    cn = n // (_NW * _NCHUNKS)
    mesh = plsc.VectorSubcoreMesh(core_axis_name="c", subcore_axis_name="s")
    run = pl.kernel(
        functools.partial(_body, cn=cn),
        out_type=jax.ShapeDtypeStruct((n * _D,), jnp.float32),
        mesh=mesh,
        scratch_types=[
            pltpu.VMEM((3 * _L,), jnp.float32),
            pltpu.VMEM((cn,), jnp.int32),
            pltpu.VMEM((cn,), jnp.int32),
            pltpu.VMEM((cn * _D,), jnp.float32),
            pltpu.VMEM((cn * _D,), jnp.float32),
            pltpu.SemaphoreType.DMA,
            pltpu.SemaphoreType.DMA,
            pltpu.SemaphoreType.DMA,
            pltpu.SemaphoreType.DMA,
        ],
        compiler_params=pltpu.CompilerParams(
            needs_layout_passes=False, use_tc_tiling_on_sc=False),
    )
    return run(x, wf)


def kernel(x, W):
    bb, s = x.shape
    n = bb * s
    wf = jnp.pad(W.reshape(-1).astype(jnp.float32), (0, 3 * _L - W.size))
    out = _lookup(x.astype(jnp.int32).reshape(-1), wf, n)
    return out.reshape(bb, s, _D)


# speedup vs baseline: 49.0011x; 49.0011x over previous
import functools

import jax
import jax.numpy as jnp
from jax import lax
from jax.experimental import pallas as pl
from jax.experimental.pallas import tpu as pltpu
from jax.experimental.pallas import tpu_sc as plsc

_NC = 2
_NS = 16
_NW = _NC * _NS
_L = 16

_D = 4
_RT = 128
_CT = 8
_NQ = 4


def _body(x_ref, w_ref, out_ref, wt_v, w0, w1, w2, w3, i0, i1, o0, o1,
          si0, si1, so0, so1, bb, s):
    wid = lax.axis_index("s") * _NC + lax.axis_index("c")
    nrt = bb // _RT // _NW
    rt0 = wid * nrt
    nct = s // _CT
    cq = s // _NQ
    cn = _RT * s

    lane = lax.iota(jnp.int32, _L)

    pltpu.sync_copy(w_ref, wt_v)
    for k in range(4):
        wt_v[pl.ds(k * _L, _L)] = wt_v[pl.ds(k * _L, _L)] * 0.5 + 1.0
    wd = (w0, w1, w2, w3)
    for d in range(_D):
        wd[d][pl.ds(0, _L)] = plsc.load_gather(wt_v, [lane * _D + d])

    ibufs = (i0, i1)
    obufs = (o0, o1)
    isems = (si0, si1)
    osems = (so0, so1)

    def in_copies(rt, b):
        return [
            pltpu.make_async_copy(
                x_ref.at[pl.ds(ct * bb * _CT + (rt0 + rt) * _CT * _RT,
                               _CT * _RT)],
                ibufs[b].at[pl.ds(ct * _CT * _RT, _CT * _RT)],
                isems[b])
            for ct in range(nct)
        ]

    def out_copy(rt, q, b):
        return pltpu.make_async_copy(
            obufs[b],
            out_ref.at[pl.ds(q * cq, cq), pl.ds((rt0 + rt) * _D * _RT,
                                                _D * _RT)],
            osems[b])

    def compute(q, ib, ob):
        iv = ibufs[ib]
        ov = obufs[ob]
        c0 = q * cq

        @plsc.parallel_loop(0, cq * (_RT // _L), unroll=8)
        def _(i):
            cl = i >> 3
            lg = i & 7
            c = c0 + cl
            base = (c // _CT) * _CT * _RT + (c % _CT) * _RT + lg * _L
            idx = plsc.load_gather(iv, [base + lane])
            for d in range(_D):
                g = plsc.load_gather(wd[d], [idx])
                ov[cl, pl.ds(d * _RT + lg * _L, _L)] = g

    for cp in in_copies(0, 0):
        cp.start()
    step = 0
    for rt in range(nrt):
        for cp in in_copies(rt, rt & 1):
            cp.wait()
        if rt + 1 < nrt:
            for cp in in_copies(rt + 1, (rt + 1) & 1):
                cp.start()
        for q in range(_NQ):
            ob = step & 1
            if step >= 2:
                out_copy((step - 2) // _NQ, (step - 2) % _NQ, ob).wait()
            compute(q, rt & 1, ob)
            out_copy(rt, q, ob).start()
            step += 1
    out_copy(nrt - 1, _NQ - 2, 0).wait()
    out_copy(nrt - 1, _NQ - 1, 1).wait()


@functools.partial(jax.jit, static_argnames=("bb", "s"))
def _lookup(x, wf, bb, s):
    cq = s // _NQ
    mesh = plsc.VectorSubcoreMesh(core_axis_name="c", subcore_axis_name="s")
    run = pl.kernel(
        functools.partial(_body, bb=bb, s=s),
        out_type=jax.ShapeDtypeStruct((s, bb * _D), jnp.float32),
        mesh=mesh,
        scratch_types=[
            pltpu.VMEM((4 * _L,), jnp.float32),
            pltpu.VMEM((_L,), jnp.float32),
            pltpu.VMEM((_L,), jnp.float32),
            pltpu.VMEM((_L,), jnp.float32),
            pltpu.VMEM((_L,), jnp.float32),
            pltpu.VMEM((_RT * s,), jnp.int32),
            pltpu.VMEM((_RT * s,), jnp.int32),
            pltpu.VMEM((cq, _D * _RT), jnp.float32),
            pltpu.VMEM((cq, _D * _RT), jnp.float32),
            pltpu.SemaphoreType.DMA,
            pltpu.SemaphoreType.DMA,
            pltpu.SemaphoreType.DMA,
            pltpu.SemaphoreType.DMA,
        ],
        compiler_params=pltpu.CompilerParams(
            needs_layout_passes=False, use_tc_tiling_on_sc=False),
    )
    return run(x, wf)


def kernel(x, W):
    bb, s = x.shape
    xt = (x.astype(jnp.int32).T
          .reshape(s // _CT, _CT, bb // _RT, _RT)
          .transpose(0, 2, 1, 3)
          .reshape(-1))
    wf = jnp.pad(W.reshape(-1).astype(jnp.float32), (0, _D * _L - W.size))
    out2d = _lookup(xt, wf, bb, s)
    return (out2d.reshape(s, bb // _RT, _D, _RT)
            .transpose(1, 3, 0, 2)
            .reshape(bb, s, _D))
